# Initial kernel scaffold; baseline (speedup 1.0000x reference)
import jax
import jax.numpy as jnp
from jax.experimental import pallas as pl


def kernel(rays_o, rays_d, occ_grid, aabb, near_far):
    raise NotImplementedError("write your pallas kernel here")


# speedup vs baseline: 200.6145x; 200.6145x over previous
import functools

import jax
import jax.numpy as jnp
from jax import lax
from jax.experimental import pallas as pl
from jax.experimental.pallas import tpu as pltpu
from jax.experimental.pallas import tpu_sc as plsc

RESO = 128
STEP = 0.01
N_STEPS = 256
N_RAYS = 16384

NW = 32
ROWS_PER_W = N_RAYS // NW
CHUNK_R = 16
N_CHUNKS = ROWS_PER_W // CHUNK_R
NVEC = N_STEPS // 16
GRID_WORDS = RESO * RESO * RESO // 32


def _sc_sample(code, grid_words, ts_tab, te_tab):
    mesh = plsc.VectorSubcoreMesh(core_axis_name="c", subcore_axis_name="s")

    @functools.partial(
        pl.kernel,
        mesh=mesh,
        compiler_params=pltpu.CompilerParams(needs_layout_passes=False),
        out_type=(
            jax.ShapeDtypeStruct((N_RAYS, N_STEPS), jnp.int32),
            jax.ShapeDtypeStruct((N_RAYS, N_STEPS), jnp.float32),
            jax.ShapeDtypeStruct((N_RAYS, N_STEPS), jnp.float32),
        ),
        scratch_types=[
            pltpu.VMEM((GRID_WORDS,), jnp.int32),
            pltpu.VMEM((N_STEPS,), jnp.float32),
            pltpu.VMEM((N_STEPS,), jnp.float32),
            pltpu.VMEM((CHUNK_R, N_STEPS), jnp.int32),
            pltpu.VMEM((CHUNK_R, N_STEPS), jnp.int32),
            pltpu.VMEM((CHUNK_R, N_STEPS), jnp.float32),
            pltpu.VMEM((CHUNK_R, N_STEPS), jnp.float32),
        ],
    )
    def k(code_hbm, grid_hbm, tst_hbm, tet_hbm, ri_hbm, ts_hbm, te_hbm,
          grid_v, tst_v, tet_v, cbuf, ribuf, tsbuf, tebuf):
        wid = lax.axis_index("s") * 2 + lax.axis_index("c")
        pltpu.sync_copy(grid_hbm, grid_v)
        pltpu.sync_copy(tst_hbm, tst_v)
        pltpu.sync_copy(tet_hbm, tet_v)
        base0 = wid * ROWS_PER_W

        def chunk_body(c, carry):
            rowbase = base0 + c * CHUNK_R
            pltpu.sync_copy(code_hbm.at[pl.ds(rowbase, CHUNK_R)], cbuf)
            for v in range(NVEC):
                sl = pl.ds(v * 16, 16)
                tsv = tst_v[sl]
                tev = tet_v[sl]

                def row_body(r, c2, sl=sl, tsv=tsv, tev=tev, rowbase=rowbase):
                    cd = cbuf[r, sl]
                    word = plsc.load_gather(grid_v, [cd >> 6])
                    occ = (word >> ((cd >> 1) & 31)) & cd & 1
                    m = occ == 1
                    ridv = jnp.full((16,), rowbase + r, dtype=jnp.int32)
                    ribuf[r, sl] = jnp.where(m, ridv, -1)
                    tsbuf[r, sl] = jnp.where(m, tsv, 0.0)
                    tebuf[r, sl] = jnp.where(m, tev, 0.0)
                    return c2

                lax.fori_loop(0, CHUNK_R, row_body, 0)
            pltpu.sync_copy(ribuf, ri_hbm.at[pl.ds(rowbase, CHUNK_R)])
            pltpu.sync_copy(tsbuf, ts_hbm.at[pl.ds(rowbase, CHUNK_R)])
            pltpu.sync_copy(tebuf, te_hbm.at[pl.ds(rowbase, CHUNK_R)])
            return carry

        lax.fori_loop(0, N_CHUNKS, chunk_body, 0)

    return k(code, grid_words, ts_tab, te_tab)


def kernel(rays_o, rays_d, occ_grid, aabb, near_far):
    d = rays_d / (jnp.linalg.norm(rays_d, axis=-1, keepdims=True) + 1e-8)
    t_mid = near_far[0] + (jnp.arange(N_STEPS, dtype=jnp.float32) + 0.5) * STEP
    pos = rays_o[:, None, :] + d[:, None, :] * t_mid[None, :, None]
    size = aabb[1] - aabb[0]
    g = (pos - aabb[0][None, None, :]) / size[None, None, :] * RESO
    idx = jnp.clip(g.astype(jnp.int32), 0, RESO - 1)
    inside = jnp.all((pos >= aabb[0][None, None, :])
                     & (pos < aabb[1][None, None, :]), axis=-1)
    widx = idx[..., 0] * 512 + idx[..., 1] * 4 + (idx[..., 2] >> 5)
    code = (widx << 6) | ((idx[..., 2] & 31) << 1) | inside.astype(jnp.int32)
    gw = occ_grid.reshape(-1, 32).astype(jnp.uint32)
    words = (gw << jnp.arange(32, dtype=jnp.uint32)[None, :]).sum(
        axis=1, dtype=jnp.uint32)
    words = lax.bitcast_convert_type(words, jnp.int32)
    tst = t_mid - 0.5 * STEP
    tet = t_mid + 0.5 * STEP
    ri, ts, te = _sc_sample(code, words, tst, tet)
    return ri, ts, te, ri >= 0
